# baseline (speedup 1.0000x reference)
import functools

import jax
import jax.numpy as jnp
from jax import lax
from jax.experimental import pallas as pl
from jax.experimental.pallas import tpu as pltpu
from jax.experimental.pallas import tpu_sc as plsc


def _make_sc_gather(num_idx, table_pad, unroll=8):
    info = plsc.get_sparse_core_info()
    nc, ns, nl = info.num_cores, info.num_subcores, info.num_lanes
    nw = nc * ns
    assert num_idx % (nw * nl * unroll) == 0
    b_per_w = num_idx // nw
    mesh = plsc.VectorSubcoreMesh(core_axis_name="c", subcore_axis_name="s")

    @functools.partial(
        pl.kernel,
        mesh=mesh,
        out_type=jax.ShapeDtypeStruct((num_idx,), jnp.float32),
        scratch_types=[
            pltpu.VMEM((table_pad,), jnp.float32),
            pltpu.VMEM((b_per_w,), jnp.int32),
            pltpu.VMEM((b_per_w,), jnp.float32),
        ],
        compiler_params=pltpu.CompilerParams(needs_layout_passes=False),
    )
    def sc_gather(table_h, idx_h, bias_h, table_v, idx_v, bias_v):
        wid = lax.axis_index("s") * nc + lax.axis_index("c")
        base = wid * b_per_w
        pltpu.sync_copy(table_h, table_v)
        pltpu.sync_copy(idx_h.at[pl.ds(base, b_per_w)], idx_v)

        def body(i, carry):
            off = i * (nl * unroll)
            for u in range(unroll):
                o = off + u * nl
                iv = idx_v[pl.ds(o, nl)]
                bias_v[pl.ds(o, nl)] = plsc.load_gather(table_v, [iv])
            return carry

        lax.fori_loop(0, b_per_w // (nl * unroll), body, 0)
        pltpu.sync_copy(bias_v, bias_h.at[pl.ds(base, b_per_w)])

    return sc_gather


def _tc_add_body(a_ref, b_ref, o_ref):
    o_ref[...] = a_ref[...] + b_ref[...][None]


def _tc_add(attn, bias2d, head_block=1):
    _, nh, n, m = attn.shape
    a3 = attn.reshape(nh, n, m)
    grid = (nh // head_block,)
    out = pl.pallas_call(
        _tc_add_body,
        grid=grid,
        in_specs=[
            pl.BlockSpec((head_block, n, m), lambda h: (h, 0, 0)),
            pl.BlockSpec((n, m), lambda h: (0, 0)),
        ],
        out_specs=pl.BlockSpec((head_block, n, m), lambda h: (h, 0, 0)),
        out_shape=jax.ShapeDtypeStruct(a3.shape, a3.dtype),
    )(a3, bias2d)
    return out.reshape(attn.shape)


def kernel(attn_weights, relative_position_index, relative_position_bias_table):
    n, m = relative_position_index.shape
    num_idx = n * m
    table_flat = relative_position_bias_table.reshape(-1)
    table_pad = ((table_flat.shape[0] + 511) // 512) * 512
    table_flat = jnp.pad(table_flat, (0, table_pad - table_flat.shape[0]))
    idx_flat = relative_position_index.reshape(-1)

    bias_flat = _make_sc_gather(num_idx, table_pad)(table_flat, idx_flat)
    bias2d = bias_flat.reshape(n, m)
    return _tc_add(attn_weights, bias2d)


# speedup vs baseline: 1.0190x; 1.0190x over previous
import functools

import jax
import jax.numpy as jnp
from jax import lax
from jax.experimental import pallas as pl
from jax.experimental.pallas import tpu as pltpu
from jax.experimental.pallas import tpu_sc as plsc


def _make_sc_gather(num_idx, table_pad, unroll=8):
    info = plsc.get_sparse_core_info()
    nc, ns, nl = info.num_cores, info.num_subcores, info.num_lanes
    nw = nc * ns
    assert num_idx % (nw * nl * unroll) == 0
    b_per_w = num_idx // nw
    mesh = plsc.VectorSubcoreMesh(core_axis_name="c", subcore_axis_name="s")

    @functools.partial(
        pl.kernel,
        mesh=mesh,
        out_type=jax.ShapeDtypeStruct((num_idx,), jnp.float32),
        scratch_types=[
            pltpu.VMEM((table_pad,), jnp.float32),
            pltpu.VMEM((b_per_w,), jnp.int32),
            pltpu.VMEM((b_per_w,), jnp.float32),
        ],
        compiler_params=pltpu.CompilerParams(needs_layout_passes=False),
    )
    def sc_gather(table_h, idx_h, bias_h, table_v, idx_v, bias_v):
        wid = lax.axis_index("s") * nc + lax.axis_index("c")
        base = wid * b_per_w
        pltpu.sync_copy(table_h, table_v)
        pltpu.sync_copy(idx_h.at[pl.ds(base, b_per_w)], idx_v)

        def body(i, carry):
            off = i * (nl * unroll)
            for u in range(unroll):
                o = off + u * nl
                iv = idx_v[pl.ds(o, nl)]
                bias_v[pl.ds(o, nl)] = plsc.load_gather(table_v, [iv])
            return carry

        lax.fori_loop(0, b_per_w // (nl * unroll), body, 0)
        pltpu.sync_copy(bias_v, bias_h.at[pl.ds(base, b_per_w)])

    return sc_gather


def _tc_add_body(a_ref, b_ref, o_ref):
    o_ref[...] = a_ref[...] + b_ref[...][None]


def _tc_add(attn, bias2d, head_block=2):
    _, nh, n, m = attn.shape
    a3 = attn.reshape(nh, n, m)
    grid = (nh // head_block,)
    out = pl.pallas_call(
        _tc_add_body,
        grid=grid,
        in_specs=[
            pl.BlockSpec((head_block, n, m), lambda h: (h, 0, 0)),
            pl.BlockSpec((n, m), lambda h: (0, 0)),
        ],
        out_specs=pl.BlockSpec((head_block, n, m), lambda h: (h, 0, 0)),
        out_shape=jax.ShapeDtypeStruct(a3.shape, a3.dtype),
    )(a3, bias2d)
    return out.reshape(attn.shape)


def kernel(attn_weights, relative_position_index, relative_position_bias_table):
    n, m = relative_position_index.shape
    num_idx = n * m
    table_flat = relative_position_bias_table.reshape(-1)
    table_pad = ((table_flat.shape[0] + 511) // 512) * 512
    table_flat = jnp.pad(table_flat, (0, table_pad - table_flat.shape[0]))
    idx_flat = relative_position_index.reshape(-1)

    bias_flat = _make_sc_gather(num_idx, table_pad)(table_flat, idx_flat)
    bias2d = bias_flat.reshape(n, m)
    return _tc_add(attn_weights, bias2d)


# speedup vs baseline: 1.1996x; 1.1772x over previous
import functools

import jax
import jax.numpy as jnp
from jax import lax
from jax.experimental import pallas as pl
from jax.experimental.pallas import tpu as pltpu
from jax.experimental.pallas import tpu_sc as plsc


def _make_sc_gather(n, m, table_pad):
    info = plsc.get_sparse_core_info()
    nc, ns, nl = info.num_cores, info.num_subcores, info.num_lanes
    nw = nc * ns
    assert n % nw == 0 and m % nl == 0
    rows_per_w = n // nw
    chunks_per_row = m // nl
    mesh = plsc.VectorSubcoreMesh(core_axis_name="c", subcore_axis_name="s")

    @functools.partial(
        pl.kernel,
        mesh=mesh,
        out_type=jax.ShapeDtypeStruct((n, m), jnp.float32),
        scratch_types=[
            pltpu.VMEM((table_pad,), jnp.float32),
            pltpu.VMEM((rows_per_w, m), jnp.int32),
            pltpu.VMEM((rows_per_w, m), jnp.float32),
        ],
        compiler_params=pltpu.CompilerParams(needs_layout_passes=False),
    )
    def sc_gather(table_h, idx_h, bias_h, table_v, idx_v, bias_v):
        wid = lax.axis_index("s") * nc + lax.axis_index("c")
        base = wid * rows_per_w
        pltpu.sync_copy(table_h, table_v)
        pltpu.sync_copy(idx_h.at[pl.ds(base, rows_per_w), :], idx_v)

        @plsc.parallel_loop(0, rows_per_w)
        def _(r):
            for c in range(chunks_per_row):
                iv = idx_v[r, pl.ds(c * nl, nl)]
                bias_v[r, pl.ds(c * nl, nl)] = plsc.load_gather(table_v, [iv])

        pltpu.sync_copy(bias_v, bias_h.at[pl.ds(base, rows_per_w), :])

    return sc_gather


def _tc_add_body(a_ref, b_ref, o_ref):
    o_ref[...] = a_ref[...] + b_ref[...][None]


def _tc_add(attn, bias2d, head_block=2):
    _, nh, n, m = attn.shape
    a3 = attn.reshape(nh, n, m)
    grid = (nh // head_block,)
    out = pl.pallas_call(
        _tc_add_body,
        grid=grid,
        in_specs=[
            pl.BlockSpec((head_block, n, m), lambda h: (h, 0, 0)),
            pl.BlockSpec((n, m), lambda h: (0, 0)),
        ],
        out_specs=pl.BlockSpec((head_block, n, m), lambda h: (h, 0, 0)),
        out_shape=jax.ShapeDtypeStruct(a3.shape, a3.dtype),
    )(a3, bias2d)
    return out.reshape(attn.shape)


def kernel(attn_weights, relative_position_index, relative_position_bias_table):
    n, m = relative_position_index.shape
    table_flat = relative_position_bias_table.reshape(-1)
    table_pad = ((table_flat.shape[0] + 511) // 512) * 512
    table_flat = jnp.pad(table_flat, (0, table_pad - table_flat.shape[0]))

    bias2d = _make_sc_gather(n, m, table_pad)(table_flat, relative_position_index)
    return _tc_add(attn_weights, bias2d)
